# baseline (speedup 1.0000x reference)
import functools
import numpy as np
import jax
import jax.numpy as jnp
from jax import lax
from jax.experimental import pallas as pl
from jax.experimental.pallas import tpu as pltpu
from jax.experimental.pallas import tpu_sc as plsc

NH = 8
CUTOFF = 5.0

NC = 2
NS = 16
NW = NC * NS


def _silu(a):
    return a * jax.nn.sigmoid(a)


def _pack_slab(x128):
    u = lax.bitcast_convert_type(x128.astype(jnp.bfloat16), jnp.uint16)
    u = u.astype(jnp.uint32)
    p = u[:, :64] | (u[:, 64:] << 16)
    return lax.bitcast_convert_type(p, jnp.float32)


def _unpack_slab(p64):
    w = lax.bitcast_convert_type(p64, jnp.int32)
    lo = lax.bitcast_convert_type(w << 16, jnp.float32)
    hi = lax.bitcast_convert_type(w & jnp.int32(-65536), jnp.float32)
    return jnp.concatenate([lo, hi], axis=1)


def _tc1a_body(xb, tb, nab, vc0, vc1, vc2, wxT, wtr, wnT, b1, w2T, b2, g, bb,
               qwT, qb, kwT, kb, vwT, vb, wv1T, wv2T, wv3T,
               q_o, sp_o, vdot_o, v3o0, v3o1, v3o2):
    f32 = jnp.float32
    h1 = _silu(jnp.dot(xb[...], wxT[...], preferred_element_type=f32)
               + tb[...] * wtr[...]
               + jnp.dot(nab[...], wnT[...], preferred_element_type=f32)
               + b1[...])
    h2 = jnp.dot(h1, w2T[...], preferred_element_type=f32) + b2[...]
    mu = jnp.mean(h2, axis=-1, keepdims=True)
    var = jnp.mean((h2 - mu) ** 2, axis=-1, keepdims=True)
    h = (h2 - mu) * jax.lax.rsqrt(var + 1e-5) * g[...] + bb[...]
    q_o[...] = jnp.dot(h, qwT[...], preferred_element_type=f32) + qb[...]
    sp_o[:, 0:64] = _pack_slab(jnp.dot(h, kwT[...], preferred_element_type=f32) + kb[...])
    v = jnp.dot(h, vwT[...], preferred_element_type=f32) + vb[...]
    sp_o[:, 64:128] = _pack_slab(v[:, 0:128])
    sp_o[:, 128:192] = _pack_slab(v[:, 128:256])
    sp_o[:, 192:256] = _pack_slab(v[:, 256:384])
    sp_o[:, 256:320] = _pack_slab(vc0[...])
    sp_o[:, 320:384] = _pack_slab(vc1[...])
    sp_o[:, 384:448] = _pack_slab(vc2[...])
    sp_o[:, 448:512] = jnp.zeros_like(v[:, 0:64])
    w1 = wv1T[...]
    w2v = wv2T[...]
    w3 = wv3T[...]
    vdot = None
    for vc, v3o in ((vc0, v3o0), (vc1, v3o1), (vc2, v3o2)):
        vcb = vc[...]
        p1 = jnp.dot(vcb, w1, preferred_element_type=f32)
        p2 = jnp.dot(vcb, w2v, preferred_element_type=f32)
        v3o[...] = jnp.dot(vcb, w3, preferred_element_type=f32)
        vdot = p1 * p2 if vdot is None else vdot + p1 * p2
    vdot_o[...] = vdot


def _tc2_body(fb, wd, bd, qg_r, spg,
              rb, d0, d1, d2, xm_o, vm0_o, vm1_o, vm2_o):
    f32 = jnp.float32
    dkv = _silu(jnp.dot(fb[...], wd[...], preferred_element_type=f32) + bd[...])
    dk = dkv[:, 0:128]
    dvx = dkv[:, 128:256]
    dv1 = dkv[:, 256:384]
    dv2 = dkv[:, 384:512]
    sg = spg[...]
    qg = qg_r[...]
    kg = _unpack_slab(sg[:, 0:64])
    vx = _unpack_slab(sg[:, 64:128])
    vm1 = _unpack_slab(sg[:, 128:192])
    vm2 = _unpack_slab(sg[:, 192:256])
    vc0 = _unpack_slab(sg[:, 256:320])
    vc1 = _unpack_slab(sg[:, 320:384])
    vc2 = _unpack_slab(sg[:, 384:448])
    hs = (lax.broadcasted_iota(jnp.int32, (128, NH), 0) // 16
          == lax.broadcasted_iota(jnp.int32, (128, NH), 1)).astype(f32)
    p = qg * kg * dk
    attn = jnp.dot(p, hs, preferred_element_type=f32)
    r = rb[...]
    cut = 0.5 * (jnp.cos(r * (np.pi / CUTOFF)) + 1.0)
    cut = jnp.where(r < CUTOFF, cut, 0.0)
    a8 = _silu(attn) * cut
    ab = jnp.dot(a8, hs.T, preferred_element_type=f32)
    xm_o[...] = vx * dvx * ab
    m1 = vm1 * dv1
    m2 = vm2 * dv2
    vm0_o[...] = vc0 * m1 + m2 * d0[...]
    vm1_o[...] = vc1 * m1 + m2 * d1[...]
    vm2_o[...] = vc2 * m1 + m2 * d2[...]


def _tc3_body(xp, vp0, vp1, vp2, v30, v31, v32, vdotb, owT, obr,
              dx_o, dvec_o):
    f32 = jnp.float32
    xa = xp[0] + xp[1]
    o = jnp.dot(xa, owT[...], preferred_element_type=f32) + obr[...]
    o1 = o[:, 0:128]
    o2 = o[:, 128:256]
    o3 = o[:, 256:384]
    dvec_o[:, 0, :] = v30[...] * o1 + (vp0[0] + vp0[1])
    dvec_o[:, 1, :] = v31[...] * o1 + (vp1[0] + vp1[1])
    dvec_o[:, 2, :] = v32[...] * o1 + (vp2[0] + vp2[1])
    dx_o[...] = vdotb[...] * o2 + o3


def _make_sc_gather(E, Bg):
    EW = E // NW
    nch = EW // Bg
    mesh = plsc.VectorSubcoreMesh(core_axis_name="c", subcore_axis_name="s",
                                  num_cores=NC, num_subcores=NS)

    @functools.partial(
        pl.kernel,
        out_type=[
            jax.ShapeDtypeStruct((E, 128), jnp.float32),
            jax.ShapeDtypeStruct((E, 512), jnp.float32),
        ],
        mesh=mesh,
        scratch_types=[
            pltpu.VMEM((EW,), jnp.int32),
            pltpu.VMEM((EW,), jnp.int32),
            pltpu.VMEM((2, Bg, 128), jnp.float32),
            pltpu.VMEM((2, Bg, 512), jnp.float32),
            pltpu.SemaphoreType.DMA,
            pltpu.SemaphoreType.DMA,
            pltpu.SemaphoreType.DMA,
            pltpu.SemaphoreType.DMA,
        ],
    )
    def sc_gather(q_hbm, sp_hbm, src_v, dst_v,
                  qg_o, spg_o, idxs, idxd, bq, bsp,
                  gsem0, gsem1, wsem0, wsem1):
        c = lax.axis_index("c")
        s = lax.axis_index("s")
        wid = s * NC + c
        ebase = wid * EW

        def fire_gathers(j, p, gsem):
            ds_d = idxd.at[pl.ds(j * Bg, Bg)]
            ds_s = idxs.at[pl.ds(j * Bg, Bg)]
            pltpu.async_copy(q_hbm.at[ds_d], bq.at[p], gsem)
            pltpu.async_copy(sp_hbm.at[ds_s], bsp.at[p], gsem)

        def wait_gathers(j, p, gsem):
            ds_d = idxd.at[pl.ds(j * Bg, Bg)]
            ds_s = idxs.at[pl.ds(j * Bg, Bg)]
            pltpu.make_async_copy(q_hbm.at[ds_d], bq.at[p], gsem).wait()
            pltpu.make_async_copy(sp_hbm.at[ds_s], bsp.at[p], gsem).wait()

        def fire_writebacks(j, p, wsem):
            gb = ebase + j * Bg
            pltpu.async_copy(bq.at[p], qg_o.at[pl.ds(gb, Bg)], wsem)
            pltpu.async_copy(bsp.at[p], spg_o.at[pl.ds(gb, Bg)], wsem)

        def drain_writebacks(j, p, wsem):
            gb = ebase + j * Bg
            pltpu.make_async_copy(bq.at[p], qg_o.at[pl.ds(gb, Bg)], wsem).wait()
            pltpu.make_async_copy(bsp.at[p], spg_o.at[pl.ds(gb, Bg)], wsem).wait()

        pltpu.sync_copy(src_v.at[pl.ds(ebase, EW)], idxs)
        pltpu.sync_copy(dst_v.at[pl.ds(ebase, EW)], idxd)
        fire_gathers(0, 0, gsem0)

        def it(i, carry):
            j0 = 2 * i
            j1 = j0 + 1

            @pl.when(i > 0)
            def _():
                drain_writebacks(j0 - 1, 1, wsem1)

            fire_gathers(j1, 1, gsem1)
            wait_gathers(j0, 0, gsem0)
            fire_writebacks(j0, 0, wsem0)

            drain_writebacks(j0, 0, wsem0)

            @pl.when(j1 + 1 < nch)
            def _():
                fire_gathers(j1 + 1, 0, gsem0)

            wait_gathers(j1, 1, gsem1)
            fire_writebacks(j1, 1, wsem1)
            return carry

        lax.fori_loop(0, nch // 2, it, 0)
        if nch % 2 == 1:
            jlast = nch - 1
            wait_gathers(jlast, 0, gsem0)
            fire_writebacks(jlast, 0, wsem0)
            drain_writebacks(jlast, 0, wsem0)
            drain_writebacks(nch - 2, 1, wsem1)
        else:
            drain_writebacks(nch - 1, 1, wsem1)

    return sc_gather


def _make_sc_scatter(E, N_PAD, Bs):
    Cs = E // Bs
    iters = (Cs + NW - 1) // NW
    rows_per_sub = N_PAD // NS
    mesh = plsc.VectorSubcoreMesh(core_axis_name="c", subcore_axis_name="s",
                                  num_cores=NC, num_subcores=NS)
    out_sh = jax.ShapeDtypeStruct((NC, N_PAD, 128), jnp.float32)

    @functools.partial(
        pl.kernel,
        out_type=[out_sh, out_sh, out_sh, out_sh],
        mesh=mesh,
        scratch_types=[
            pltpu.VMEM((1, Bs), jnp.int32),
            pltpu.VMEM((Bs, 128), jnp.float32),
            pltpu.VMEM_SHARED((N_PAD, 128), jnp.float32),
        ],
    )
    def sc_scatter(m0, m1, m2, m3, dst_r, zeros_hbm,
                   o0, o1, o2, o3, idxb, msgb, acc):
        c = lax.axis_index("c")
        s = lax.axis_index("s")
        wid = s * NC + c
        row0 = s * rows_per_sub
        msgs = (m0, m1, m2, m3)
        outs = (o0, o1, o2, o3)
        for g in range(4):
            mg = msgs[g]
            pltpu.sync_copy(zeros_hbm.at[pl.ds(row0, rows_per_sub)],
                            acc.at[pl.ds(row0, rows_per_sub)])
            plsc.subcore_barrier()

            def it(i, carry):
                ch = wid + i * NW

                @pl.when(ch < Cs)
                def _():
                    pltpu.sync_copy(dst_r.at[ch], idxb)
                    pltpu.sync_copy(mg.at[pl.ds(ch * Bs, Bs)], msgb)
                    pltpu.sync_copy(msgb, acc.at[idxb.at[0]], add=True)

                return carry

            lax.fori_loop(0, iters, it, 0)
            plsc.subcore_barrier()
            pltpu.sync_copy(acc.at[pl.ds(row0, rows_per_sub)],
                            outs[g].at[c, pl.ds(row0, rows_per_sub)])
            plsc.subcore_barrier()

    return sc_scatter


def kernel(x, vec, edge_index, r_ij, f_ij, d_ij, t, node_attr,
           mix_w1, mix_b1, mix_w2, mix_b2, ln_g, ln_b,
           q_w, q_b, k_w, k_b, v_w, v_b, o_w, o_b, vec_w,
           dk_w, dk_b, dv_w, dv_b):
    f32 = jnp.float32
    N, H = x.shape
    E = r_ij.shape[0]
    NRBF = f_ij.shape[1]
    TN = 1000
    TE = 1280
    Bg = 80
    Bs = 256

    idx = np.arange(3 * H)
    perm = (idx % H // 16) * 48 + (idx // H) * 16 + (idx % 16)
    v_wp = v_w[perm]
    v_bp = v_b[perm]
    dv_wp = dv_w[perm]
    dv_bp = dv_b[perm]

    row = lambda b: b.reshape(1, -1)
    wxT = mix_w1[:, :H].T
    wtr = mix_w1[:, H].reshape(1, H)
    wnT = mix_w1[:, H + 1:].T
    wd = jnp.concatenate([dk_w, dv_wp], axis=0).T
    bd = jnp.concatenate([dk_b, dv_bp]).reshape(1, 512)

    vec_c = [vec[:, c, :] for c in range(3)]
    src = edge_index[0]
    dst = edge_index[1]
    dst_s = dst.reshape(E // Bs, 1, Bs)
    N_PAD = ((N // NS + 7) // 8 * 8) * NS
    r2 = r_ij.reshape(E, 1)
    d0 = d_ij[:, 0].reshape(E, 1)
    d1 = d_ij[:, 1].reshape(E, 1)
    d2 = d_ij[:, 2].reshape(E, 1)

    full = lambda sh: pl.BlockSpec(sh, lambda i: (0,) * len(sh))
    nblk = lambda w: pl.BlockSpec((TN, w), lambda i: (i, 0))

    q, sp, vdot, v30, v31, v32 = pl.pallas_call(
        _tc1a_body,
        grid=(N // TN,),
        in_specs=[
            nblk(H), nblk(1), nblk(H), nblk(H), nblk(H), nblk(H),
            full((H, H)), full((1, H)), full((H, H)), full((1, H)),
            full((H, H)), full((1, H)), full((1, H)), full((1, H)),
            full((H, H)), full((1, H)), full((H, H)), full((1, H)),
            full((H, 3 * H)), full((1, 3 * H)),
            full((H, H)), full((H, H)), full((H, H)),
        ],
        out_specs=[nblk(H), nblk(512),
                   nblk(H), nblk(H), nblk(H), nblk(H)],
        out_shape=[
            jax.ShapeDtypeStruct((N, H), f32),
            jax.ShapeDtypeStruct((N, 512), f32),
            jax.ShapeDtypeStruct((N, H), f32),
            jax.ShapeDtypeStruct((N, H), f32),
            jax.ShapeDtypeStruct((N, H), f32),
            jax.ShapeDtypeStruct((N, H), f32),
        ],
    )(x, t, node_attr, vec_c[0], vec_c[1], vec_c[2],
      wxT, wtr, wnT, row(mix_b1), mix_w2.T, row(mix_b2), row(ln_g), row(ln_b),
      q_w.T, row(q_b), k_w.T, row(k_b), v_wp.T, row(v_bp),
      vec_w[:H].T, vec_w[H:2 * H].T, vec_w[2 * H:].T)

    qg, spg = _make_sc_gather(E, Bg)(q, sp, src, dst)

    eblk = lambda w, j: pl.BlockSpec((TE, w), lambda i, j=j: (i, j))
    xm, vm0, vm1, vm2 = pl.pallas_call(
        _tc2_body,
        grid=(E // TE,),
        in_specs=[
            pl.BlockSpec((TE, NRBF), lambda i: (i, 0)),
            full((NRBF, 512)), full((1, 512)),
            eblk(128, 0), eblk(512, 0),
            eblk(1, 0), eblk(1, 0), eblk(1, 0), eblk(1, 0),
        ],
        out_specs=[eblk(128, 0)] * 4,
        out_shape=[jax.ShapeDtypeStruct((E, 128), f32)] * 4,
    )(f_ij, wd, bd, qg, spg, r2, d0, d1, d2)

    zeros_hbm = jnp.zeros((N_PAD, 128), dtype=f32)
    xp, vp0, vp1, vp2 = _make_sc_scatter(E, N_PAD, Bs)(
        xm, vm0, vm1, vm2, dst_s, zeros_hbm)

    pblk = pl.BlockSpec((NC, TN, 128), lambda i: (0, i, 0))
    dx, dvec = pl.pallas_call(
        _tc3_body,
        grid=(N // TN,),
        in_specs=[pblk, pblk, pblk, pblk, nblk(H), nblk(H), nblk(H), nblk(H),
                  full((H, 3 * H)), full((1, 3 * H))],
        out_specs=[nblk(H), pl.BlockSpec((TN, 3, H), lambda i: (i, 0, 0))],
        out_shape=[jax.ShapeDtypeStruct((N, H), f32),
                   jax.ShapeDtypeStruct((N, 3, H), f32)],
    )(xp, vp0, vp1, vp2, v30, v31, v32, vdot, o_w.T, row(o_b))

    return (dx, dvec)


# speedup vs baseline: 1.0801x; 1.0801x over previous
import functools
import numpy as np
import jax
import jax.numpy as jnp
from jax import lax
from jax.experimental import pallas as pl
from jax.experimental.pallas import tpu as pltpu
from jax.experimental.pallas import tpu_sc as plsc

NH = 8
CUTOFF = 5.0

NC = 2
NS = 16
NW = NC * NS


def _silu(a):
    return a * jax.nn.sigmoid(a)


def _pack_slab(x128):
    u = lax.bitcast_convert_type(x128.astype(jnp.bfloat16), jnp.uint16)
    u = u.astype(jnp.uint32)
    p = u[:, :64] | (u[:, 64:] << 16)
    return lax.bitcast_convert_type(p, jnp.float32)


def _unpack_slab(p64):
    w = lax.bitcast_convert_type(p64, jnp.int32)
    lo = lax.bitcast_convert_type(w << 16, jnp.float32)
    hi = lax.bitcast_convert_type(w & jnp.int32(-65536), jnp.float32)
    return jnp.concatenate([lo, hi], axis=1)


def _tc1a_body(xb, tb, nab, vc0, vc1, vc2, wxT, wtr, wnT, b1, w2T, b2, g, bb,
               qwT, qb, kwT, kb, vwT, vb, wv1T, wv2T, wv3T,
               q_o, sp_o, vdot_o, v3o0, v3o1, v3o2):
    f32 = jnp.float32
    h1 = _silu(jnp.dot(xb[...], wxT[...], preferred_element_type=f32)
               + tb[...] * wtr[...]
               + jnp.dot(nab[...], wnT[...], preferred_element_type=f32)
               + b1[...])
    h2 = jnp.dot(h1, w2T[...], preferred_element_type=f32) + b2[...]
    mu = jnp.mean(h2, axis=-1, keepdims=True)
    var = jnp.mean((h2 - mu) ** 2, axis=-1, keepdims=True)
    h = (h2 - mu) * jax.lax.rsqrt(var + 1e-5) * g[...] + bb[...]
    q_o[...] = jnp.dot(h, qwT[...], preferred_element_type=f32) + qb[...]
    sp_o[:, 0:64] = _pack_slab(jnp.dot(h, kwT[...], preferred_element_type=f32) + kb[...])
    v = jnp.dot(h, vwT[...], preferred_element_type=f32) + vb[...]
    sp_o[:, 64:128] = _pack_slab(v[:, 0:128])
    sp_o[:, 128:192] = _pack_slab(v[:, 128:256])
    sp_o[:, 192:256] = _pack_slab(v[:, 256:384])
    sp_o[:, 256:320] = _pack_slab(vc0[...])
    sp_o[:, 320:384] = _pack_slab(vc1[...])
    sp_o[:, 384:448] = _pack_slab(vc2[...])
    sp_o[:, 448:512] = jnp.zeros_like(v[:, 0:64])
    w1 = wv1T[...]
    w2v = wv2T[...]
    w3 = wv3T[...]
    vdot = None
    for vc, v3o in ((vc0, v3o0), (vc1, v3o1), (vc2, v3o2)):
        vcb = vc[...]
        p1 = jnp.dot(vcb, w1, preferred_element_type=f32)
        p2 = jnp.dot(vcb, w2v, preferred_element_type=f32)
        v3o[...] = jnp.dot(vcb, w3, preferred_element_type=f32)
        vdot = p1 * p2 if vdot is None else vdot + p1 * p2
    vdot_o[...] = vdot


def _tc2_body(fb, wd, bd, qg_r, spg,
              rb, d0, d1, d2, xm_o, vm0_o, vm1_o, vm2_o):
    f32 = jnp.float32
    dkv = _silu(jnp.dot(fb[...], wd[...], preferred_element_type=f32) + bd[...])
    dk = dkv[:, 0:128]
    dvx = dkv[:, 128:256]
    dv1 = dkv[:, 256:384]
    dv2 = dkv[:, 384:512]
    sg = spg[...]
    qg = qg_r[...]
    kg = _unpack_slab(sg[:, 0:64])
    vx = _unpack_slab(sg[:, 64:128])
    vm1 = _unpack_slab(sg[:, 128:192])
    vm2 = _unpack_slab(sg[:, 192:256])
    vc0 = _unpack_slab(sg[:, 256:320])
    vc1 = _unpack_slab(sg[:, 320:384])
    vc2 = _unpack_slab(sg[:, 384:448])
    hs = (lax.broadcasted_iota(jnp.int32, (128, NH), 0) // 16
          == lax.broadcasted_iota(jnp.int32, (128, NH), 1)).astype(f32)
    p = qg * kg * dk
    attn = jnp.dot(p, hs, preferred_element_type=f32)
    r = rb[...]
    cut = 0.5 * (jnp.cos(r * (np.pi / CUTOFF)) + 1.0)
    cut = jnp.where(r < CUTOFF, cut, 0.0)
    a8 = _silu(attn) * cut
    ab = jnp.dot(a8, hs.T, preferred_element_type=f32)
    xm_o[...] = vx * dvx * ab
    m1 = vm1 * dv1
    m2 = vm2 * dv2
    vm0_o[...] = vc0 * m1 + m2 * d0[...]
    vm1_o[...] = vc1 * m1 + m2 * d1[...]
    vm2_o[...] = vc2 * m1 + m2 * d2[...]


def _tc3_body(xp, vp0, vp1, vp2, v30, v31, v32, vdotb, owT, obr,
              dx_o, dvec_o):
    f32 = jnp.float32
    xa = xp[0] + xp[1]
    o = jnp.dot(xa, owT[...], preferred_element_type=f32) + obr[...]
    o1 = o[:, 0:128]
    o2 = o[:, 128:256]
    o3 = o[:, 256:384]
    dvec_o[:, 0, :] = v30[...] * o1 + (vp0[0] + vp0[1])
    dvec_o[:, 1, :] = v31[...] * o1 + (vp1[0] + vp1[1])
    dvec_o[:, 2, :] = v32[...] * o1 + (vp2[0] + vp2[1])
    dx_o[...] = vdotb[...] * o2 + o3


def _make_sc_gather(E, Bg):
    EW = E // NW
    nch = EW // Bg
    mesh = plsc.VectorSubcoreMesh(core_axis_name="c", subcore_axis_name="s",
                                  num_cores=NC, num_subcores=NS)

    @functools.partial(
        pl.kernel,
        out_type=[
            jax.ShapeDtypeStruct((E, 128), jnp.float32),
            jax.ShapeDtypeStruct((E, 512), jnp.float32),
        ],
        mesh=mesh,
        scratch_types=[
            pltpu.VMEM((EW,), jnp.int32),
            pltpu.VMEM((EW,), jnp.int32),
            pltpu.VMEM((2, Bg, 128), jnp.float32),
            pltpu.VMEM((2, Bg, 512), jnp.float32),
            pltpu.SemaphoreType.DMA,
            pltpu.SemaphoreType.DMA,
            pltpu.SemaphoreType.DMA,
            pltpu.SemaphoreType.DMA,
        ],
    )
    def sc_gather(q_hbm, sp_hbm, src_v, dst_v,
                  qg_o, spg_o, idxs, idxd, bq, bsp,
                  gsem0, gsem1, wsem0, wsem1):
        c = lax.axis_index("c")
        s = lax.axis_index("s")
        wid = s * NC + c
        ebase = wid * EW

        def fire_gathers(j, p, gsem):
            ds_d = idxd.at[pl.ds(j * Bg, Bg)]
            ds_s = idxs.at[pl.ds(j * Bg, Bg)]
            pltpu.async_copy(q_hbm.at[ds_d], bq.at[p], gsem)
            pltpu.async_copy(sp_hbm.at[ds_s], bsp.at[p], gsem)

        def wait_gathers(j, p, gsem):
            ds_d = idxd.at[pl.ds(j * Bg, Bg)]
            ds_s = idxs.at[pl.ds(j * Bg, Bg)]
            pltpu.make_async_copy(q_hbm.at[ds_d], bq.at[p], gsem).wait()
            pltpu.make_async_copy(sp_hbm.at[ds_s], bsp.at[p], gsem).wait()

        def fire_writebacks(j, p, wsem):
            gb = ebase + j * Bg
            pltpu.async_copy(bq.at[p], qg_o.at[pl.ds(gb, Bg)], wsem)
            pltpu.async_copy(bsp.at[p], spg_o.at[pl.ds(gb, Bg)], wsem)

        def drain_writebacks(j, p, wsem):
            gb = ebase + j * Bg
            pltpu.make_async_copy(bq.at[p], qg_o.at[pl.ds(gb, Bg)], wsem).wait()
            pltpu.make_async_copy(bsp.at[p], spg_o.at[pl.ds(gb, Bg)], wsem).wait()

        pltpu.sync_copy(src_v.at[pl.ds(ebase, EW)], idxs)
        pltpu.sync_copy(dst_v.at[pl.ds(ebase, EW)], idxd)
        fire_gathers(0, 0, gsem0)

        def it(i, carry):
            j0 = 2 * i
            j1 = j0 + 1

            @pl.when(i > 0)
            def _():
                drain_writebacks(j0 - 1, 1, wsem1)

            fire_gathers(j1, 1, gsem1)
            wait_gathers(j0, 0, gsem0)
            fire_writebacks(j0, 0, wsem0)

            drain_writebacks(j0, 0, wsem0)

            @pl.when(j1 + 1 < nch)
            def _():
                fire_gathers(j1 + 1, 0, gsem0)

            wait_gathers(j1, 1, gsem1)
            fire_writebacks(j1, 1, wsem1)
            return carry

        lax.fori_loop(0, nch // 2, it, 0)
        if nch % 2 == 1:
            jlast = nch - 1
            wait_gathers(jlast, 0, gsem0)
            fire_writebacks(jlast, 0, wsem0)
            drain_writebacks(jlast, 0, wsem0)
            drain_writebacks(nch - 2, 1, wsem1)
        else:
            drain_writebacks(nch - 1, 1, wsem1)

    return sc_gather


def _make_sc_scatter(E, N_PAD, Bs):
    EW = E // NW
    nch = EW // Bs
    rows_per_sub = N_PAD // NS
    mesh = plsc.VectorSubcoreMesh(core_axis_name="c", subcore_axis_name="s",
                                  num_cores=NC, num_subcores=NS)
    out_sh = jax.ShapeDtypeStruct((NC, N_PAD, 128), jnp.float32)

    @functools.partial(
        pl.kernel,
        out_type=[out_sh, out_sh, out_sh, out_sh],
        mesh=mesh,
        scratch_types=[
            pltpu.VMEM((nch, Bs), jnp.int32),
            pltpu.VMEM((2, Bs, 128), jnp.float32),
            pltpu.VMEM_SHARED((N_PAD, 128), jnp.float32),
            pltpu.SemaphoreType.DMA,
            pltpu.SemaphoreType.DMA,
            pltpu.SemaphoreType.DMA,
            pltpu.SemaphoreType.DMA,
        ],
    )
    def sc_scatter(m0, m1, m2, m3, dst_w, zeros_hbm,
                   o0, o1, o2, o3, idxb, msgb, acc,
                   lsem0, lsem1, ssem0, ssem1):
        c = lax.axis_index("c")
        s = lax.axis_index("s")
        wid = s * NC + c
        ebase = wid * EW
        row0 = s * rows_per_sub
        msgs = (m0, m1, m2, m3)
        outs = (o0, o1, o2, o3)
        pltpu.sync_copy(dst_w.at[wid], idxb)
        for g in range(4):
            mg = msgs[g]

            def load(j, p, lsem):
                pltpu.async_copy(mg.at[pl.ds(ebase + j * Bs, Bs)],
                                 msgb.at[p], lsem)

            def wait_load(j, p, lsem):
                pltpu.make_async_copy(mg.at[pl.ds(ebase + j * Bs, Bs)],
                                      msgb.at[p], lsem).wait()

            def fire_add(j, p, ssem):
                pltpu.async_copy(msgb.at[p], acc.at[idxb.at[j]], ssem,
                                 add=True)

            def drain_add(j, p, ssem):
                pltpu.make_async_copy(msgb.at[p], acc.at[idxb.at[j]],
                                      ssem).wait()

            pltpu.sync_copy(zeros_hbm.at[pl.ds(row0, rows_per_sub)],
                            acc.at[pl.ds(row0, rows_per_sub)])
            plsc.subcore_barrier()
            load(0, 0, lsem0)

            def it(i, carry):
                j0 = 2 * i
                j1 = j0 + 1

                @pl.when(i > 0)
                def _():
                    drain_add(j0 - 1, 1, ssem1)

                load(j1, 1, lsem1)
                wait_load(j0, 0, lsem0)
                fire_add(j0, 0, ssem0)

                drain_add(j0, 0, ssem0)

                @pl.when(j1 + 1 < nch)
                def _():
                    load(j1 + 1, 0, lsem0)

                wait_load(j1, 1, lsem1)
                fire_add(j1, 1, ssem1)
                return carry

            lax.fori_loop(0, nch // 2, it, 0)
            if nch % 2 == 1:
                jlast = nch - 1
                wait_load(jlast, 0, lsem0)
                fire_add(jlast, 0, ssem0)
                drain_add(jlast, 0, ssem0)
                drain_add(nch - 2, 1, ssem1)
            else:
                drain_add(nch - 1, 1, ssem1)
            plsc.subcore_barrier()
            pltpu.sync_copy(acc.at[pl.ds(row0, rows_per_sub)],
                            outs[g].at[c, pl.ds(row0, rows_per_sub)])
            plsc.subcore_barrier()

    return sc_scatter


def kernel(x, vec, edge_index, r_ij, f_ij, d_ij, t, node_attr,
           mix_w1, mix_b1, mix_w2, mix_b2, ln_g, ln_b,
           q_w, q_b, k_w, k_b, v_w, v_b, o_w, o_b, vec_w,
           dk_w, dk_b, dv_w, dv_b):
    f32 = jnp.float32
    N, H = x.shape
    E = r_ij.shape[0]
    NRBF = f_ij.shape[1]
    TN = 1000
    TE = 1280
    Bg = 80
    Bs = 80

    idx = np.arange(3 * H)
    perm = (idx % H // 16) * 48 + (idx // H) * 16 + (idx % 16)
    v_wp = v_w[perm]
    v_bp = v_b[perm]
    dv_wp = dv_w[perm]
    dv_bp = dv_b[perm]

    row = lambda b: b.reshape(1, -1)
    wxT = mix_w1[:, :H].T
    wtr = mix_w1[:, H].reshape(1, H)
    wnT = mix_w1[:, H + 1:].T
    wd = jnp.concatenate([dk_w, dv_wp], axis=0).T
    bd = jnp.concatenate([dk_b, dv_bp]).reshape(1, 512)

    vec_c = [vec[:, c, :] for c in range(3)]
    src = edge_index[0]
    dst = edge_index[1]
    dst_s = dst.reshape(NW, E // (NW * Bs), Bs)
    N_PAD = ((N // NS + 7) // 8 * 8) * NS
    r2 = r_ij.reshape(E, 1)
    d0 = d_ij[:, 0].reshape(E, 1)
    d1 = d_ij[:, 1].reshape(E, 1)
    d2 = d_ij[:, 2].reshape(E, 1)

    full = lambda sh: pl.BlockSpec(sh, lambda i: (0,) * len(sh))
    nblk = lambda w: pl.BlockSpec((TN, w), lambda i: (i, 0))

    q, sp, vdot, v30, v31, v32 = pl.pallas_call(
        _tc1a_body,
        grid=(N // TN,),
        in_specs=[
            nblk(H), nblk(1), nblk(H), nblk(H), nblk(H), nblk(H),
            full((H, H)), full((1, H)), full((H, H)), full((1, H)),
            full((H, H)), full((1, H)), full((1, H)), full((1, H)),
            full((H, H)), full((1, H)), full((H, H)), full((1, H)),
            full((H, 3 * H)), full((1, 3 * H)),
            full((H, H)), full((H, H)), full((H, H)),
        ],
        out_specs=[nblk(H), nblk(512),
                   nblk(H), nblk(H), nblk(H), nblk(H)],
        out_shape=[
            jax.ShapeDtypeStruct((N, H), f32),
            jax.ShapeDtypeStruct((N, 512), f32),
            jax.ShapeDtypeStruct((N, H), f32),
            jax.ShapeDtypeStruct((N, H), f32),
            jax.ShapeDtypeStruct((N, H), f32),
            jax.ShapeDtypeStruct((N, H), f32),
        ],
    )(x, t, node_attr, vec_c[0], vec_c[1], vec_c[2],
      wxT, wtr, wnT, row(mix_b1), mix_w2.T, row(mix_b2), row(ln_g), row(ln_b),
      q_w.T, row(q_b), k_w.T, row(k_b), v_wp.T, row(v_bp),
      vec_w[:H].T, vec_w[H:2 * H].T, vec_w[2 * H:].T)

    qg, spg = _make_sc_gather(E, Bg)(q, sp, src, dst)

    eblk = lambda w, j: pl.BlockSpec((TE, w), lambda i, j=j: (i, j))
    xm, vm0, vm1, vm2 = pl.pallas_call(
        _tc2_body,
        grid=(E // TE,),
        in_specs=[
            pl.BlockSpec((TE, NRBF), lambda i: (i, 0)),
            full((NRBF, 512)), full((1, 512)),
            eblk(128, 0), eblk(512, 0),
            eblk(1, 0), eblk(1, 0), eblk(1, 0), eblk(1, 0),
        ],
        out_specs=[eblk(128, 0)] * 4,
        out_shape=[jax.ShapeDtypeStruct((E, 128), f32)] * 4,
    )(f_ij, wd, bd, qg, spg, r2, d0, d1, d2)

    zeros_hbm = jnp.zeros((N_PAD, 128), dtype=f32)
    xp, vp0, vp1, vp2 = _make_sc_scatter(E, N_PAD, Bs)(
        xm, vm0, vm1, vm2, dst_s, zeros_hbm)

    pblk = pl.BlockSpec((NC, TN, 128), lambda i: (0, i, 0))
    dx, dvec = pl.pallas_call(
        _tc3_body,
        grid=(N // TN,),
        in_specs=[pblk, pblk, pblk, pblk, nblk(H), nblk(H), nblk(H), nblk(H),
                  full((H, 3 * H)), full((1, 3 * H))],
        out_specs=[nblk(H), pl.BlockSpec((TN, 3, H), lambda i: (i, 0, 0))],
        out_shape=[jax.ShapeDtypeStruct((N, H), f32),
                   jax.ShapeDtypeStruct((N, 3, H), f32)],
    )(xp, vp0, vp1, vp2, v30, v31, v32, vdot, o_w.T, row(o_b))

    return (dx, dvec)
